# baseline (speedup 1.0000x reference)
import jax
import jax.numpy as jnp
from jax import lax
from jax.experimental import pallas as pl
from jax.experimental.pallas import tpu as pltpu
from jax.experimental.pallas import tpu_sc as plsc

DIM = 64
NF = 50
B = 16384
NC = 2
NS = 16
NW = NC * NS
BPW = B // NW
CHUNK = 128
NCHUNK = BPW // CHUNK


def _lanesum(x):
    idx = jnp.arange(16, dtype=jnp.int32)
    dnums = lax.GatherDimensionNumbers(
        offset_dims=(), collapsed_slice_dims=(0,), start_index_map=(0,))
    for sh in (8, 4, 2, 1):
        perm = jnp.bitwise_xor(idx, sh)
        x = x + lax.gather(x, perm[:, None], dimension_numbers=dnums,
                           slice_sizes=(1,),
                           mode=lax.GatherScatterMode.PROMISE_IN_BOUNDS)
    return x


def _rsqrt16(x):
    i = plsc.bitcast(x, jnp.int32)
    i = jnp.int32(0x5F3759DF) - jnp.right_shift(i, 1)
    y = plsc.bitcast(i, jnp.float32)
    half = x * jnp.float32(0.5)
    for _ in range(3):
        y = y * (jnp.float32(1.5) - half * y * y)
    return y


def _body(idx_hbm, e_hbm, r_hbm, wk_hbm, out_hbm,
          idx_v, hv, rv, tv, wkv, score_v, sem):
    wid = lax.axis_index("s") * NC + lax.axis_index("c")
    base_row = wid * NCHUNK

    pltpu.sync_copy(idx_hbm.at[:, pl.ds(base_row, NCHUNK)], idx_v)
    pltpu.sync_copy(wk_hbm, wkv)

    copies = []
    for c in range(NCHUNK):
        copies.append(pltpu.async_copy(
            e_hbm.at[idx_v.at[0, c]], hv.at[pl.ds(c * CHUNK, CHUNK)], sem))
        copies.append(pltpu.async_copy(
            r_hbm.at[idx_v.at[1, c]], rv.at[pl.ds(c * CHUNK, CHUNK)], sem))
        copies.append(pltpu.async_copy(
            e_hbm.at[idx_v.at[2, c]], tv.at[pl.ds(c * CHUNK, CHUNK)], sem))
    for cp in copies:
        cp.wait()

    lane0 = jnp.arange(16, dtype=jnp.int32) == 0

    def triple(i, carry):
        h = [hv[i, pl.ds(16 * k, 16)] for k in range(4)]
        r = [rv[i, pl.ds(16 * k, 16)] for k in range(4)]
        t = [tv[i, pl.ds(16 * k, 16)] for k in range(4)]

        def inv_norm(x):
            ssq = x[0] * x[0] + x[1] * x[1] + x[2] * x[2] + x[3] * x[3]
            s = _lanesum(ssq)
            return _rsqrt16(jnp.maximum(s, jnp.float32(1e-24)))

        ih, ir, it = inv_norm(h), inv_norm(r), inv_norm(t)
        h = [x * ih for x in h]
        r = [x * ir for x in r]
        t = [x * it for x in t]

        acc = [jnp.zeros((16,), jnp.float32) for _ in range(4)]
        for f in range(NF):
            ka = wkv[f, pl.ds(DIM, 16)]
            kb = wkv[f, pl.ds(DIM + 16, 16)]
            kc = wkv[f, pl.ds(DIM + 32, 16)]
            for k in range(4):
                z = h[k] * ka + r[k] * kb + t[k] * kc
                z = jnp.maximum(z, jnp.float32(0.0))
                acc[k] = acc[k] + z * wkv[f, pl.ds(16 * k, 16)]
        tot = -_lanesum(acc[0] + acc[1] + acc[2] + acc[3])
        plsc.store_scatter(score_v, [jnp.full((16,), i, jnp.int32)], tot,
                           mask=lane0)
        return carry

    lax.fori_loop(0, BPW, triple, 0)
    pltpu.sync_copy(score_v, out_hbm.at[pl.ds(wid * BPW, BPW)])


def kernel(T, E_table, R_table, kernel, fc_W):
    idx = T.T.reshape(3, B // CHUNK, CHUNK).astype(jnp.int32)
    k3 = kernel[:, 0, 0, :]
    kbt = jnp.repeat(k3, 16, axis=1).astype(jnp.float32)
    W = fc_W.reshape(NF, DIM)
    wk = jnp.concatenate(
        [W, kbt, jnp.zeros((NF, 128 - DIM - 48), jnp.float32)], axis=1)

    mesh = plsc.VectorSubcoreMesh(core_axis_name="c", subcore_axis_name="s")
    run = pl.kernel(
        _body,
        out_type=jax.ShapeDtypeStruct((B,), jnp.float32),
        mesh=mesh,
        compiler_params=pltpu.CompilerParams(needs_layout_passes=False,
                                             use_tc_tiling_on_sc=False),
        scratch_types=[
            pltpu.VMEM((3, NCHUNK, CHUNK), jnp.int32),
            pltpu.VMEM((BPW, DIM), jnp.float32),
            pltpu.VMEM((BPW, DIM), jnp.float32),
            pltpu.VMEM((BPW, DIM), jnp.float32),
            pltpu.VMEM((NF, 128), jnp.float32),
            pltpu.VMEM((BPW,), jnp.float32),
            pltpu.SemaphoreType.DMA,
        ],
    )
    return run(idx, E_table, R_table, wk)
